# baseline (speedup 1.0000x reference)
import jax
import jax.numpy as jnp
from jax.experimental import pallas as pl
from jax.experimental.pallas import tpu as pltpu

BLK = 1024
EPS = 1e-5


def _y_body(xc_ref, w1t_ref, y_ref):
    y_ref[...] = jax.lax.dot_general(
        xc_ref[...], w1t_ref[...], (((1,), (0,)), ((), ())),
        preferred_element_type=jnp.float32)


def _knn_body(pf_ref, pcT_ref, y_ref, w1b_ref, xf_ref, out_ref, st_ref):
    i = pl.program_id(0)
    nc = pcT_ref.shape[1]

    @pl.when(i == 0)
    def _():
        st_ref[...] = jnp.zeros_like(st_ref)

    d2 = None
    for d in range(3):
        diff = pf_ref[:, d:d + 1] - pcT_ref[d:d + 1, :]
        sq = diff * diff
        d2 = sq if d2 is None else d2 + sq

    ms = []
    sels = []
    for j in range(3):
        m = jnp.min(d2, axis=1, keepdims=True)
        sel = d2 == m
        ms.append(m)
        sels.append(sel)
        if j < 2:
            d2 = jnp.where(sel, jnp.float32(jnp.inf), d2)

    w0 = 1.0 / jnp.maximum(ms[0], 1e-16)
    w1 = 1.0 / jnp.maximum(ms[1], 1e-16)
    w2 = 1.0 / jnp.maximum(ms[2], 1e-16)
    den = w0 + w1 + w2
    s_mat = (jnp.where(sels[0], w0 / den, 0.0)
             + jnp.where(sels[1], w1 / den, 0.0)
             + jnp.where(sels[2], w2 / den, 0.0))

    h_interp = jax.lax.dot_general(
        s_mat, y_ref[...], (((1,), (0,)), ((), ())),
        preferred_element_type=jnp.float32)
    h_fine = jax.lax.dot_general(
        xf_ref[...], w1b_ref[...], (((1,), (0,)), ((), ())),
        preferred_element_type=jnp.float32)
    h = h_interp + h_fine
    out_ref[...] = h.astype(jnp.bfloat16)

    s1 = jnp.sum(h, axis=0, keepdims=True)
    s2 = jnp.sum(h * h, axis=0, keepdims=True)
    st_ref[0:1, :] += s1
    st_ref[1:2, :] += s2


def _mlp2_body(h_ref, st_ref, g_ref, b_ref, w2_ref, out_ref, st2_ref, *, n):
    i = pl.program_id(0)
    mu = st_ref[0:1, :] * (1.0 / n)
    var = st_ref[1:2, :] * (1.0 / n) - mu * mu
    inv = jax.lax.rsqrt(var + EPS)
    h = (h_ref[...].astype(jnp.float32) - mu) * (inv * g_ref[...]) + b_ref[...]
    h = jnp.maximum(h, 0.0)
    h2 = jax.lax.dot_general(
        h, w2_ref[...], (((1,), (0,)), ((), ())),
        preferred_element_type=jnp.float32)
    out_ref[...] = h2.astype(jnp.bfloat16)

    @pl.when(i == 0)
    def _():
        st2_ref[...] = jnp.zeros_like(st2_ref)

    st2_ref[0:1, :] += jnp.sum(h2, axis=0, keepdims=True)
    st2_ref[1:2, :] += jnp.sum(h2 * h2, axis=0, keepdims=True)


def _bn2_body(h_ref, st_ref, g_ref, b_ref, out_ref, *, n):
    mu = st_ref[0:1, :] * (1.0 / n)
    var = st_ref[1:2, :] * (1.0 / n) - mu * mu
    inv = jax.lax.rsqrt(var + EPS)
    h = (h_ref[...].astype(jnp.float32) - mu) * (inv * g_ref[...]) + b_ref[...]
    out_ref[...] = jnp.maximum(h, 0.0)


def kernel(x_coarse, pos_coarse, batch_coarse, x_fine, pos_fine, batch_fine,
           W1, gamma1, beta1, W2, gamma2, beta2):
    nc, cc = x_coarse.shape
    nf, cf = x_fine.shape
    out_dim = W1.shape[1]
    grid = nf // BLK

    pcT = pos_coarse.T
    g1 = gamma1.reshape(1, -1)
    b1 = beta1.reshape(1, -1)
    g2 = gamma2.reshape(1, -1)
    b2 = beta2.reshape(1, -1)

    full = lambda *s: pl.BlockSpec(s, lambda i: (0,) * len(s))
    row_blk = lambda c: pl.BlockSpec((BLK, c), lambda i: (i, 0))

    y = pl.pallas_call(
        _y_body,
        grid=(1,),
        in_specs=[full(nc, cc), full(cc, out_dim)],
        out_specs=full(nc, out_dim),
        out_shape=jax.ShapeDtypeStruct((nc, out_dim), jnp.float32),
    )(x_coarse, W1)

    h1, st1 = pl.pallas_call(
        _knn_body,
        grid=(grid,),
        in_specs=[row_blk(3), full(3, nc), full(nc, out_dim),
                  pl.BlockSpec((cf, out_dim), lambda i: (cc // cf, 0)),
                  row_blk(cf)],
        out_specs=[row_blk(out_dim), full(8, out_dim)],
        out_shape=[jax.ShapeDtypeStruct((nf, out_dim), jnp.bfloat16),
                   jax.ShapeDtypeStruct((8, out_dim), jnp.float32)],
        compiler_params=pltpu.CompilerParams(
            dimension_semantics=("arbitrary",)),
    )(pos_fine, pcT, y, W1, x_fine)

    import functools
    h2, st2 = pl.pallas_call(
        functools.partial(_mlp2_body, n=float(nf)),
        grid=(grid,),
        in_specs=[row_blk(out_dim), full(8, out_dim), full(1, out_dim),
                  full(1, out_dim), full(out_dim, out_dim)],
        out_specs=[row_blk(out_dim), full(8, out_dim)],
        out_shape=[jax.ShapeDtypeStruct((nf, out_dim), jnp.bfloat16),
                   jax.ShapeDtypeStruct((8, out_dim), jnp.float32)],
        compiler_params=pltpu.CompilerParams(
            dimension_semantics=("arbitrary",)),
    )(h1, st1, g1, b1, W2)

    out = pl.pallas_call(
        functools.partial(_bn2_body, n=float(nf)),
        grid=(grid,),
        in_specs=[row_blk(out_dim), full(8, out_dim), full(1, out_dim),
                  full(1, out_dim)],
        out_specs=row_blk(out_dim),
        out_shape=jax.ShapeDtypeStruct((nf, out_dim), jnp.float32),
        compiler_params=pltpu.CompilerParams(
            dimension_semantics=("arbitrary",)),
    )(h2, st2, g2, b2)

    return out
